# Initial kernel scaffold; baseline (speedup 1.0000x reference)
import jax
import jax.numpy as jnp
from jax.experimental import pallas as pl


def kernel(x, duration, max_len):
    raise NotImplementedError("write your pallas kernel here")


# speedup vs baseline: 25.4901x; 25.4901x over previous
import functools

import jax
import jax.numpy as jnp
from jax import lax
from jax.experimental import pallas as pl
from jax.experimental.pallas import tpu as pltpu
from jax.experimental.pallas import tpu_sc as plsc

_B, _S, _D, _T = 16, 512, 512, 2048
_SP = _S + 1
_L = 16
_NW = 32
_ROWS_W = (_B * _T) // _NW
_CHUNK = 64
_NCHUNK = _ROWS_W // _CHUNK


def _build_expand():
    mesh = plsc.VectorSubcoreMesh(core_axis_name="c", subcore_axis_name="s")

    @functools.partial(
        pl.kernel,
        mesh=mesh,
        compiler_params=pltpu.CompilerParams(needs_layout_passes=False),
        out_type=(
            jax.ShapeDtypeStruct((_B * _T, _D), jnp.float32),
            jax.ShapeDtypeStruct((_B,), jnp.int32),
        ),
        scratch_types=[
            pltpu.VMEM((_S,), jnp.int32),
            pltpu.VMEM((_S,), jnp.int32),
            pltpu.VMEM((_ROWS_W,), jnp.int32),
            pltpu.VMEM((_CHUNK, _D), jnp.float32),
            pltpu.VMEM((_B * _S,), jnp.int32),
            pltpu.VMEM((_B,), jnp.int32),
            pltpu.VMEM((_L,), jnp.int32),
            pltpu.SemaphoreType.DMA,
        ],
    )
    def expand(dur_hbm, xflat_hbm, ml_hbm, out_hbm, len_hbm,
               dur_v, cum_v, gidx_v, buf_v, dur_all_v, len_v, ml_v, sem):
        cid = lax.axis_index("c")
        sid = lax.axis_index("s")
        wid = sid * 2 + cid
        b = wid // 2
        t0 = (wid % 2) * (_T // 2)

        pltpu.sync_copy(dur_hbm.at[pl.ds(b * _S, _S)], dur_v)
        pltpu.sync_copy(ml_hbm, ml_v)
        max_len = ml_v[...][0]

        def cum_step(i, carry):
            v = jnp.clip(dur_v[pl.ds(i * _L, _L)], 0, 300)
            cum_v[pl.ds(i * _L, _L)] = plsc.cumsum(v) + carry
            return carry + jnp.sum(v)

        total = lax.fori_loop(0, _S // _L, cum_step, jnp.int32(0))
        len_b = jnp.minimum(total, max_len)

        zero_row = b * _SP + _S

        def idx_step(i, _):
            t = t0 + i * _L + lax.iota(jnp.int32, _L)
            j = jnp.zeros((_L,), jnp.int32)
            for s in (256, 128, 64, 32, 16, 8, 4, 2, 1):
                m = j + s
                probe = plsc.load_gather(cum_v, [jnp.minimum(m - 1, _S - 1)])
                j = jnp.where(probe <= t, m, j)
            tok = jnp.minimum(j, _S - 1)
            gidx_v[pl.ds(i * _L, _L)] = jnp.where(t < len_b, b * _SP + tok,
                                                  zero_row)
            return 0

        lax.fori_loop(0, _ROWS_W // _L, idx_step, 0)

        @pl.when(wid == 0)
        def _():
            pltpu.sync_copy(dur_hbm, dur_all_v)
            lane_base = lax.iota(jnp.int32, _L) * _S

            def len_step(i, acc):
                v = plsc.load_gather(dur_all_v, [lane_base + i])
                return acc + jnp.clip(v, 0, 300)

            tot = lax.fori_loop(0, _S, len_step, jnp.zeros((_L,), jnp.int32))
            len_v[...] = jnp.minimum(tot, ml_v[...])
            pltpu.sync_copy(len_v, len_hbm)

        row0 = wid * _ROWS_W

        def chunk_step(k, _):
            r = k * _CHUNK
            pltpu.async_copy(xflat_hbm.at[gidx_v.at[pl.ds(r, _CHUNK)]],
                             buf_v, sem).wait()
            pltpu.sync_copy(buf_v, out_hbm.at[pl.ds(row0 + r, _CHUNK)])
            return 0

        lax.fori_loop(0, _NCHUNK, chunk_step, 0)

    return expand


_EXPAND = _build_expand()


def kernel(x, duration, max_len):
    x_pad = jnp.concatenate(
        [x, jnp.zeros((_B, 1, _D), x.dtype)], axis=1).reshape(_B * _SP, _D)
    dur_flat = duration.astype(jnp.int32).reshape(_B * _S)
    ml = jnp.full((_L,), max_len, jnp.int32)
    out_flat, lengths = _EXPAND(dur_flat, x_pad, ml)
    return out_flat.reshape(_B, _T, _D), lengths


# speedup vs baseline: 26.5910x; 1.0432x over previous
import functools

import jax
import jax.numpy as jnp
from jax import lax
from jax.experimental import pallas as pl
from jax.experimental.pallas import tpu as pltpu
from jax.experimental.pallas import tpu_sc as plsc

_B, _S, _D, _T = 16, 512, 512, 2048
_SP = _S + 1
_L = 16
_NW = 32
_ROWS_W = (_B * _T) // _NW
_CHUNK = 64
_NCHUNK = _ROWS_W // _CHUNK


def _build_expand():
    mesh = plsc.VectorSubcoreMesh(core_axis_name="c", subcore_axis_name="s")

    @functools.partial(
        pl.kernel,
        mesh=mesh,
        compiler_params=pltpu.CompilerParams(needs_layout_passes=False),
        out_type=(
            jax.ShapeDtypeStruct((_B * _T, _D), jnp.float32),
            jax.ShapeDtypeStruct((_B,), jnp.int32),
        ),
        scratch_types=[
            pltpu.VMEM((_S,), jnp.int32),
            pltpu.VMEM((_S,), jnp.int32),
            pltpu.VMEM((_ROWS_W,), jnp.int32),
            pltpu.VMEM((_CHUNK, _D), jnp.float32),
            pltpu.VMEM((_CHUNK, _D), jnp.float32),
            pltpu.VMEM((_B * _S,), jnp.int32),
            pltpu.VMEM((_B,), jnp.int32),
            pltpu.VMEM((_L,), jnp.int32),
            pltpu.SemaphoreType.DMA,
            pltpu.SemaphoreType.DMA,
        ],
    )
    def expand(dur_hbm, xflat_hbm, ml_hbm, out_hbm, len_hbm,
               dur_v, cum_v, gidx_v, buf0, buf1, dur_all_v, len_v, ml_v,
               gsem, wsem):
        cid = lax.axis_index("c")
        sid = lax.axis_index("s")
        wid = sid * 2 + cid
        b = wid // 2
        t0 = (wid % 2) * (_T // 2)

        pltpu.sync_copy(dur_hbm.at[pl.ds(b * _S, _S)], dur_v)
        pltpu.sync_copy(ml_hbm, ml_v)
        max_len = ml_v[...][0]

        def cum_step(i, carry):
            v = jnp.clip(dur_v[pl.ds(i * _L, _L)], 0, 300)
            cum_v[pl.ds(i * _L, _L)] = plsc.cumsum(v) + carry
            return carry + jnp.sum(v)

        total = lax.fori_loop(0, _S // _L, cum_step, jnp.int32(0))
        len_b = jnp.minimum(total, max_len)

        zero_row = b * _SP + _S

        def idx_step(i, _):
            t = t0 + i * _L + lax.iota(jnp.int32, _L)
            j = jnp.zeros((_L,), jnp.int32)
            for s in (256, 128, 64, 32, 16, 8, 4, 2, 1):
                m = j + s
                probe = plsc.load_gather(cum_v, [jnp.minimum(m - 1, _S - 1)])
                j = jnp.where(probe <= t, m, j)
            tok = jnp.minimum(j, _S - 1)
            gidx_v[pl.ds(i * _L, _L)] = jnp.where(t < len_b, b * _SP + tok,
                                                  zero_row)
            return 0

        lax.fori_loop(0, _ROWS_W // _L, idx_step, 0)

        row0 = wid * _ROWS_W

        def gstart(c, buf):
            pltpu.async_copy(
                xflat_hbm.at[gidx_v.at[pl.ds(c * _CHUNK, _CHUNK)]], buf, gsem)

        def gwait(buf):
            pltpu.make_async_copy(
                xflat_hbm.at[pl.ds(0, _CHUNK)], buf, gsem).wait()

        def wstart(c, buf):
            pltpu.async_copy(
                buf, out_hbm.at[pl.ds(row0 + c * _CHUNK, _CHUNK)], wsem)

        def wdrain(buf):
            pltpu.make_async_copy(
                buf, out_hbm.at[pl.ds(row0, _CHUNK)], wsem).wait()

        gstart(0, buf0)

        @pl.when(wid == 0)
        def _():
            pltpu.sync_copy(dur_hbm, dur_all_v)
            lane_base = lax.iota(jnp.int32, _L) * _S

            def len_step(i, acc):
                v = plsc.load_gather(dur_all_v, [lane_base + i])
                return acc + jnp.clip(v, 0, 300)

            tot = lax.fori_loop(0, _S, len_step, jnp.zeros((_L,), jnp.int32))
            len_v[...] = jnp.minimum(tot, ml_v[...])
            pltpu.sync_copy(len_v, len_hbm)

        def super_step(kk, _):
            c0 = 2 * kk
            c1 = c0 + 1
            gwait(buf0)

            @pl.when(kk > 0)
            def _():
                wdrain(buf1)

            gstart(c1, buf1)
            wstart(c0, buf0)
            gwait(buf1)
            wdrain(buf0)

            @pl.when(kk < _NCHUNK // 2 - 1)
            def _():
                gstart(c0 + 2, buf0)

            wstart(c1, buf1)
            return 0

        lax.fori_loop(0, _NCHUNK // 2, super_step, 0)
        wdrain(buf1)

    return expand


_EXPAND = _build_expand()


def kernel(x, duration, max_len):
    x_pad = jnp.concatenate(
        [x, jnp.zeros((_B, 1, _D), x.dtype)], axis=1).reshape(_B * _SP, _D)
    dur_flat = duration.astype(jnp.int32).reshape(_B * _S)
    ml = jnp.full((_L,), max_len, jnp.int32)
    out_flat, lengths = _EXPAND(dur_flat, x_pad, ml)
    return out_flat.reshape(_B, _T, _D), lengths


# speedup vs baseline: 28.9190x; 1.0875x over previous
import functools

import jax
import jax.numpy as jnp
from jax import lax
from jax.experimental import pallas as pl
from jax.experimental.pallas import tpu as pltpu
from jax.experimental.pallas import tpu_sc as plsc

_B, _S, _D, _T = 16, 512, 512, 2048
_SP = _S + 1
_L = 16
_NW = 32
_ROWS_W = (_B * _T) // _NW
_CHUNK = 64
_NCHUNK = _ROWS_W // _CHUNK


def _build_expand():
    mesh = plsc.VectorSubcoreMesh(core_axis_name="c", subcore_axis_name="s")

    @functools.partial(
        pl.kernel,
        mesh=mesh,
        compiler_params=pltpu.CompilerParams(needs_layout_passes=False),
        out_type=(
            jax.ShapeDtypeStruct((_B * _T, _D), jnp.float32),
            jax.ShapeDtypeStruct((_B,), jnp.int32),
        ),
        scratch_types=[
            pltpu.VMEM((_S,), jnp.int32),
            pltpu.VMEM((_S,), jnp.int32),
            pltpu.VMEM((_ROWS_W,), jnp.int32),
            pltpu.VMEM((_CHUNK, _D), jnp.float32),
            pltpu.VMEM((_CHUNK, _D), jnp.float32),
            pltpu.VMEM((_B * _S,), jnp.int32),
            pltpu.VMEM((_B,), jnp.int32),
            pltpu.VMEM((_L,), jnp.int32),
            pltpu.SemaphoreType.DMA,
            pltpu.SemaphoreType.DMA,
        ],
    )
    def expand(dur_hbm, xflat_hbm, ml_hbm, out_hbm, len_hbm,
               dur_v, cum_v, gidx_v, buf0, buf1, dur_all_v, len_v, ml_v,
               gsem, wsem):
        cid = lax.axis_index("c")
        sid = lax.axis_index("s")
        wid = cid * (_NW // 2) + sid
        b = wid // 2
        t0 = (wid % 2) * (_T // 2)

        pltpu.sync_copy(dur_hbm.at[pl.ds(b * _S, _S)], dur_v)
        pltpu.sync_copy(ml_hbm, ml_v)
        max_len = ml_v[...][0]

        def cum_step(i, carry):
            v = jnp.clip(dur_v[pl.ds(i * _L, _L)], 0, 300)
            cum_v[pl.ds(i * _L, _L)] = plsc.cumsum(v) + carry
            return carry + jnp.sum(v)

        total = lax.fori_loop(0, _S // _L, cum_step, jnp.int32(0))
        len_b = jnp.minimum(total, max_len)

        zero_row = b * _SP + _S

        def idx_step(i, _):
            t = t0 + i * _L + lax.iota(jnp.int32, _L)
            j = jnp.zeros((_L,), jnp.int32)
            for s in (256, 128, 64, 32, 16, 8, 4, 2, 1):
                m = j + s
                probe = plsc.load_gather(cum_v, [jnp.minimum(m - 1, _S - 1)])
                j = jnp.where(probe <= t, m, j)
            tok = jnp.minimum(j, _S - 1)
            gidx_v[pl.ds(i * _L, _L)] = jnp.where(t < len_b, b * _SP + tok,
                                                  zero_row)
            return 0

        lax.fori_loop(0, _ROWS_W // _L, idx_step, 0)

        row0 = wid * _ROWS_W

        def gstart(c, buf):
            pltpu.async_copy(
                xflat_hbm.at[gidx_v.at[pl.ds(c * _CHUNK, _CHUNK)]], buf, gsem)

        def gwait(buf):
            pltpu.make_async_copy(
                xflat_hbm.at[pl.ds(0, _CHUNK)], buf, gsem).wait()

        def wstart(c, buf):
            pltpu.async_copy(
                buf, out_hbm.at[pl.ds(row0 + c * _CHUNK, _CHUNK)], wsem)

        def wdrain(buf):
            pltpu.make_async_copy(
                buf, out_hbm.at[pl.ds(row0, _CHUNK)], wsem).wait()

        gstart(0, buf0)

        @pl.when(wid == 0)
        def _():
            pltpu.sync_copy(dur_hbm, dur_all_v)
            lane_base = lax.iota(jnp.int32, _L) * _S

            def len_step(i, acc):
                v = plsc.load_gather(dur_all_v, [lane_base + i])
                return acc + jnp.clip(v, 0, 300)

            tot = lax.fori_loop(0, _S, len_step, jnp.zeros((_L,), jnp.int32))
            len_v[...] = jnp.minimum(tot, ml_v[...])
            pltpu.sync_copy(len_v, len_hbm)

        def super_step(kk, _):
            c0 = 2 * kk
            c1 = c0 + 1
            gwait(buf0)

            @pl.when(kk > 0)
            def _():
                wdrain(buf1)

            gstart(c1, buf1)
            wstart(c0, buf0)
            gwait(buf1)
            wdrain(buf0)

            @pl.when(kk < _NCHUNK // 2 - 1)
            def _():
                gstart(c0 + 2, buf0)

            wstart(c1, buf1)
            return 0

        lax.fori_loop(0, _NCHUNK // 2, super_step, 0)
        wdrain(buf1)

    return expand


_EXPAND = _build_expand()


def kernel(x, duration, max_len):
    x_pad = jnp.concatenate(
        [x, jnp.zeros((_B, 1, _D), x.dtype)], axis=1).reshape(_B * _SP, _D)
    dur_flat = duration.astype(jnp.int32).reshape(_B * _S)
    ml = jnp.full((_L,), max_len, jnp.int32)
    out_flat, lengths = _EXPAND(dur_flat, x_pad, ml)
    return out_flat.reshape(_B, _T, _D), lengths


# speedup vs baseline: 41.8302x; 1.4465x over previous
import functools

import jax
import jax.numpy as jnp
from jax import lax
from jax.experimental import pallas as pl
from jax.experimental.pallas import tpu as pltpu
from jax.experimental.pallas import tpu_sc as plsc

_B, _S, _D, _T = 16, 512, 512, 2048
_L = 16
_NW = 32
_ROWS_W = (_B * _T) // _NW
_CHUNK = 64
_NCHUNK = _ROWS_W // _CHUNK


def _build_expand():
    mesh = plsc.VectorSubcoreMesh(core_axis_name="c", subcore_axis_name="s")

    @functools.partial(
        pl.kernel,
        mesh=mesh,
        compiler_params=pltpu.CompilerParams(needs_layout_passes=False),
        out_type=(
            jax.ShapeDtypeStruct((_B * _T, _D), jnp.float32),
            jax.ShapeDtypeStruct((_B,), jnp.int32),
        ),
        scratch_types=[
            pltpu.VMEM((_S,), jnp.int32),
            pltpu.VMEM((_S,), jnp.int32),
            pltpu.VMEM((_ROWS_W,), jnp.int32),
            pltpu.VMEM((_CHUNK, _D), jnp.float32),
            pltpu.VMEM((_CHUNK, _D), jnp.float32),
            pltpu.VMEM((_CHUNK, _D), jnp.float32),
            pltpu.VMEM((_B * _S,), jnp.int32),
            pltpu.VMEM((_B,), jnp.int32),
            pltpu.VMEM((_L,), jnp.int32),
            pltpu.SemaphoreType.DMA,
            pltpu.SemaphoreType.DMA,
            pltpu.SemaphoreType.DMA,
        ],
    )
    def expand(dur_hbm, xflat_hbm, zc_hbm, ml_hbm, out_hbm, len_hbm,
               dur_v, cum_v, gidx_v, buf0, buf1, zbuf, dur_all_v, len_v,
               ml_v, gsem, wsem, zsem):
        cid = lax.axis_index("c")
        sid = lax.axis_index("s")
        wid = cid * (_NW // 2) + sid
        b = wid // 2
        t0 = (wid % 2) * (_T // 2)

        pltpu.sync_copy(dur_hbm.at[pl.ds(b * _S, _S)], dur_v)
        pltpu.sync_copy(ml_hbm, ml_v)
        pltpu.sync_copy(zc_hbm, zbuf)
        max_len = ml_v[...][0]

        def cum_step(i, carry):
            v = jnp.clip(dur_v[pl.ds(i * _L, _L)], 0, 300)
            cum_v[pl.ds(i * _L, _L)] = plsc.cumsum(v) + carry
            return carry + jnp.sum(v)

        total = lax.fori_loop(0, _S // _L, cum_step, jnp.int32(0))
        len_b = jnp.minimum(total, max_len)
        len_local = jnp.clip(len_b - t0, 0, _ROWS_W)

        def idx_step(i, _):
            t = t0 + i * _L + lax.iota(jnp.int32, _L)
            j = jnp.zeros((_L,), jnp.int32)
            for s in (256, 128, 64, 32, 16, 8, 4, 2, 1):
                m = j + s
                probe = plsc.load_gather(cum_v, [jnp.minimum(m - 1, _S - 1)])
                j = jnp.where(probe <= t, m, j)
            tok = jnp.where(t < len_b, jnp.minimum(j, _S - 1), 0)
            gidx_v[pl.ds(i * _L, _L)] = b * _S + tok
            return 0

        lax.fori_loop(0, _ROWS_W // _L, idx_step, 0)

        row0 = wid * _ROWS_W

        def valid(c):
            return c * _CHUNK < len_local

        def gstart(c, buf):
            @pl.when(valid(c))
            def _():
                pltpu.async_copy(
                    xflat_hbm.at[gidx_v.at[pl.ds(c * _CHUNK, _CHUNK)]],
                    buf, gsem)

        def gwait(c, buf):
            @pl.when(valid(c))
            def _():
                pltpu.make_async_copy(
                    xflat_hbm.at[pl.ds(0, _CHUNK)], buf, gsem).wait()

        def wstart(c, buf):
            @pl.when(valid(c))
            def _():
                pltpu.async_copy(
                    buf, out_hbm.at[pl.ds(row0 + c * _CHUNK, _CHUNK)], wsem)

        def wdrain(c, buf):
            @pl.when(valid(c))
            def _():
                pltpu.make_async_copy(
                    buf, out_hbm.at[pl.ds(row0, _CHUNK)], wsem).wait()

        rem0 = len_local % _CHUNK
        cb = len_local // _CHUNK

        def ztail(c, buf):
            @pl.when((rem0 > 0) & (c == cb))
            def _():
                def zrow(r, _):
                    @pl.when(r >= rem0)
                    def _():
                        for jj in range(_D // _L):
                            buf[r, pl.ds(jj * _L, _L)] = (
                                jnp.zeros((_L,), jnp.float32))
                    return 0

                lax.fori_loop(0, _CHUNK, zrow, 0)

        nk = (_ROWS_W - len_local) // _CHUNK

        def zfill(k, _):
            off = pl.multiple_of(_ROWS_W - _CHUNK * (k + 1), _CHUNK)
            pltpu.async_copy(zbuf, out_hbm.at[pl.ds(row0 + off, _CHUNK)],
                             zsem)
            return 0

        lax.fori_loop(0, nk, zfill, 0)

        gstart(0, buf0)

        @pl.when(wid == 0)
        def _():
            pltpu.sync_copy(dur_hbm, dur_all_v)
            lane_base = lax.iota(jnp.int32, _L) * _S

            def len_step(i, acc):
                v = plsc.load_gather(dur_all_v, [lane_base + i])
                return acc + jnp.clip(v, 0, 300)

            tot = lax.fori_loop(0, _S, len_step, jnp.zeros((_L,), jnp.int32))
            len_v[...] = jnp.minimum(tot, ml_v[...])
            pltpu.sync_copy(len_v, len_hbm)

        def super_step(kk, _):
            c0 = 2 * kk
            c1 = c0 + 1
            gwait(c0, buf0)

            @pl.when(kk > 0)
            def _():
                wdrain(c1 - 2, buf1)

            gstart(c1, buf1)
            ztail(c0, buf0)
            wstart(c0, buf0)
            gwait(c1, buf1)
            ztail(c1, buf1)
            wdrain(c0, buf0)

            @pl.when(kk < _NCHUNK // 2 - 1)
            def _():
                gstart(c0 + 2, buf0)

            wstart(c1, buf1)
            return 0

        lax.fori_loop(0, _NCHUNK // 2, super_step, 0)
        wdrain(_NCHUNK - 1, buf1)

        def zdrain(k, _):
            pltpu.make_async_copy(
                zbuf, out_hbm.at[pl.ds(row0, _CHUNK)], zsem).wait()
            return 0

        lax.fori_loop(0, nk, zdrain, 0)

    return expand


_EXPAND = _build_expand()


def kernel(x, duration, max_len):
    x_flat = x.reshape(_B * _S, _D)
    dur_flat = duration.astype(jnp.int32).reshape(_B * _S)
    zc = jnp.zeros((_CHUNK, _D), jnp.float32)
    ml = jnp.full((_L,), max_len, jnp.int32)
    out_flat, lengths = _EXPAND(dur_flat, x_flat, zc, ml)
    return out_flat.reshape(_B, _T, _D), lengths


# speedup vs baseline: 45.1462x; 1.0793x over previous
import functools

import jax
import jax.numpy as jnp
from jax import lax
from jax.experimental import pallas as pl
from jax.experimental.pallas import tpu as pltpu
from jax.experimental.pallas import tpu_sc as plsc

_B, _S, _D, _T = 16, 512, 512, 2048
_L = 16
_NW = 32
_ROWS_W = (_B * _T) // _NW
_CHUNK = 64
_NCHUNK = _ROWS_W // _CHUNK
_ZROWS = 32


def _build_expand():
    mesh = plsc.VectorSubcoreMesh(core_axis_name="c", subcore_axis_name="s")

    @functools.partial(
        pl.kernel,
        mesh=mesh,
        compiler_params=pltpu.CompilerParams(needs_layout_passes=False),
        out_type=(
            jax.ShapeDtypeStruct((_B * _T, _D), jnp.float32),
            jax.ShapeDtypeStruct((_B,), jnp.int32),
        ),
        scratch_types=[
            pltpu.VMEM((_S,), jnp.int32),
            pltpu.VMEM((_S,), jnp.int32),
            pltpu.VMEM((_ROWS_W,), jnp.int32),
            pltpu.VMEM((_CHUNK, _D), jnp.float32),
            pltpu.VMEM((_CHUNK, _D), jnp.float32),
            pltpu.VMEM((_CHUNK, _D), jnp.float32),
            pltpu.VMEM((_ZROWS, _D), jnp.float32),
            pltpu.VMEM((_B * _S,), jnp.int32),
            pltpu.VMEM((_B,), jnp.int32),
            pltpu.VMEM((_L,), jnp.int32),
            pltpu.SemaphoreType.DMA,
            pltpu.SemaphoreType.DMA,
            pltpu.SemaphoreType.DMA,
            pltpu.SemaphoreType.DMA,
            pltpu.SemaphoreType.DMA,
            pltpu.SemaphoreType.DMA,
            pltpu.SemaphoreType.DMA,
        ],
    )
    def expand(dur_hbm, xflat_hbm, zc_hbm, ml_hbm, out_hbm, len_hbm,
               dur_v, cum_v, gidx_v, buf0, buf1, buf2, zbuf, dur_all_v,
               len_v, ml_v, gs0, gs1, gs2, ws0, ws1, ws2, zsem):
        bufs = (buf0, buf1, buf2)
        gsems = (gs0, gs1, gs2)
        wsems = (ws0, ws1, ws2)
        cid = lax.axis_index("c")
        sid = lax.axis_index("s")
        wid = cid * (_NW // 2) + sid
        b = wid // 2
        t0 = (wid % 2) * (_T // 2)

        pltpu.sync_copy(dur_hbm.at[pl.ds(b * _S, _S)], dur_v)
        pltpu.sync_copy(ml_hbm, ml_v)
        pltpu.sync_copy(zc_hbm, zbuf)
        max_len = ml_v[...][0]

        def cum_step(i, carry):
            v = jnp.clip(dur_v[pl.ds(i * _L, _L)], 0, 300)
            cum_v[pl.ds(i * _L, _L)] = plsc.cumsum(v) + carry
            return carry + jnp.sum(v)

        total = lax.fori_loop(0, _S // _L, cum_step, jnp.int32(0))
        len_b = jnp.minimum(total, max_len)
        len_local = jnp.clip(len_b - t0, 0, _ROWS_W)

        def idx_step(i, _):
            t = t0 + i * _L + lax.iota(jnp.int32, _L)
            j = jnp.zeros((_L,), jnp.int32)
            for s in (256, 128, 64, 32, 16, 8, 4, 2, 1):
                m = j + s
                probe = plsc.load_gather(cum_v, [jnp.minimum(m - 1, _S - 1)])
                j = jnp.where(probe <= t, m, j)
            tok = jnp.where(t < len_b, jnp.minimum(j, _S - 1), 0)
            gidx_v[pl.ds(i * _L, _L)] = b * _S + tok
            return 0

        lax.fori_loop(0, _ROWS_W // _L, idx_step, 0)

        row0 = wid * _ROWS_W

        def valid(c):
            return c * _CHUNK < len_local

        def gstart(c):
            @pl.when(valid(c))
            def _():
                pltpu.async_copy(
                    xflat_hbm.at[gidx_v.at[pl.ds(c * _CHUNK, _CHUNK)]],
                    bufs[c % 3], gsems[c % 3])

        def gwait(c):
            @pl.when(valid(c))
            def _():
                pltpu.make_async_copy(
                    xflat_hbm.at[pl.ds(0, _CHUNK)], bufs[c % 3],
                    gsems[c % 3]).wait()

        def wstart(c):
            @pl.when(valid(c))
            def _():
                pltpu.async_copy(
                    bufs[c % 3], out_hbm.at[pl.ds(row0 + c * _CHUNK, _CHUNK)],
                    wsems[c % 3])

        def wdrain(c):
            @pl.when(valid(c))
            def _():
                pltpu.make_async_copy(
                    bufs[c % 3], out_hbm.at[pl.ds(row0, _CHUNK)],
                    wsems[c % 3]).wait()

        rem0 = len_local % _CHUNK
        cb = len_local // _CHUNK

        def ztail(c, buf):
            @pl.when((rem0 > 0) & (c == cb))
            def _():
                def zrow(r, _):
                    @pl.when(r >= rem0)
                    def _():
                        for jj in range(_D // _L):
                            buf[r, pl.ds(jj * _L, _L)] = (
                                jnp.zeros((_L,), jnp.float32))
                    return 0

                lax.fori_loop(0, _CHUNK, zrow, 0)

        nk = (_ROWS_W - len_local) // _ZROWS

        def zfill(k, _):
            off = pl.multiple_of(_ROWS_W - _ZROWS * (k + 1), _ZROWS)
            pltpu.async_copy(zbuf, out_hbm.at[pl.ds(row0 + off, _ZROWS)],
                             zsem)
            return 0

        lax.fori_loop(0, nk, zfill, 0)

        gstart(0)
        gstart(1)

        @pl.when(wid == 0)
        def _():
            pltpu.sync_copy(dur_hbm, dur_all_v)
            lane_base = lax.iota(jnp.int32, _L) * _S

            def len_step(i, acc):
                v = plsc.load_gather(dur_all_v, [lane_base + i])
                return acc + jnp.clip(v, 0, 300)

            tot = lax.fori_loop(0, _S, len_step, jnp.zeros((_L,), jnp.int32))
            len_v[...] = jnp.minimum(tot, ml_v[...])
            pltpu.sync_copy(len_v, len_hbm)

        for c in range(_NCHUNK):
            gwait(c)
            if c + 2 < _NCHUNK:
                if c >= 1:
                    wdrain(c - 1)
                gstart(c + 2)
            ztail(c, bufs[c % 3])
            wstart(c)
        wdrain(_NCHUNK - 3)
        wdrain(_NCHUNK - 2)
        wdrain(_NCHUNK - 1)

        def zdrain(k, _):
            pltpu.make_async_copy(
                zbuf, out_hbm.at[pl.ds(row0, _ZROWS)], zsem).wait()
            return 0

        lax.fori_loop(0, nk, zdrain, 0)

    return expand


_EXPAND = _build_expand()


def kernel(x, duration, max_len):
    x_flat = x.reshape(_B * _S, _D)
    dur_flat = duration.astype(jnp.int32).reshape(_B * _S)
    zc = jnp.zeros((_ZROWS, _D), jnp.float32)
    ml = jnp.full((_L,), max_len, jnp.int32)
    out_flat, lengths = _EXPAND(dur_flat, x_flat, zc, ml)
    return out_flat.reshape(_B, _T, _D), lengths
